# Initial kernel scaffold; baseline (speedup 1.0000x reference)
import jax
import jax.numpy as jnp
from jax.experimental import pallas as pl


def kernel(embedding_table, label_indices):
    raise NotImplementedError("write your pallas kernel here")


# speedup vs baseline: 1.6691x; 1.6691x over previous
import functools

import jax
import jax.numpy as jnp
from jax.experimental import pallas as pl
from jax.experimental.pallas import tpu as pltpu

_BLOCK_ROWS = 128


def _round_up(x: int, m: int) -> int:
    return ((x + m - 1) // m) * m


def _gather_block_kernel(idx_ref, table_hbm, out_ref, sem, *, block_rows):
    base = pl.program_id(0) * block_rows
    for r in range(block_rows):
        row = idx_ref[base + r]
        pltpu.make_async_copy(
            table_hbm.at[pl.ds(row, 1), :],
            out_ref.at[pl.ds(r, 1), :],
            sem,
        ).start()
    pltpu.make_async_copy(
        table_hbm.at[pl.ds(0, block_rows), :],
        out_ref.at[pl.ds(0, block_rows), :],
        sem,
    ).wait()


def kernel(embedding_table, label_indices):
    nc, d = embedding_table.shape
    n = int(label_indices.shape[0])

    idx = jnp.clip(label_indices.astype(jnp.int32), 0, nc - 1)

    block_rows = min(_BLOCK_ROWS, _round_up(max(n, 1), 8))
    n_pad = _round_up(max(n, 1), block_rows)
    if n_pad != n:
        idx = jnp.pad(idx, (0, n_pad - n))

    gather_fn = functools.partial(_gather_block_kernel, block_rows=block_rows)
    grid_spec = pltpu.PrefetchScalarGridSpec(
        num_scalar_prefetch=1,
        grid=(n_pad // block_rows,),
        in_specs=[pl.BlockSpec(memory_space=pl.ANY)],
        out_specs=pl.BlockSpec((block_rows, d), lambda i, idx_ref: (i, 0)),
        scratch_shapes=[pltpu.SemaphoreType.DMA],
    )
    out = pl.pallas_call(
        gather_fn,
        out_shape=jax.ShapeDtypeStruct((n_pad, d), embedding_table.dtype),
        grid_spec=grid_spec,
        compiler_params=pltpu.CompilerParams(
            dimension_semantics=("parallel",),
        ),
    )(idx, embedding_table)
    return out[:n]


# speedup vs baseline: 1.6727x; 1.0022x over previous
import functools

import jax
import jax.numpy as jnp
from jax.experimental import pallas as pl
from jax.experimental.pallas import tpu as pltpu

_BLOCK_ROWS = 128


def _round_up(x: int, m: int) -> int:
    return ((x + m - 1) // m) * m


def _gather_block_kernel(idx_ref, table_hbm, out_ref, sem, *, block_rows):
    base = pl.program_id(0) * block_rows
    for r in range(block_rows):
        row = idx_ref[base + r]
        pltpu.make_async_copy(
            table_hbm.at[pl.ds(row, 1), :],
            out_ref.at[pl.ds(r, 1), :],
            sem,
        ).start(priority=r % 2)
    pltpu.make_async_copy(
        table_hbm.at[pl.ds(0, block_rows), :],
        out_ref.at[pl.ds(0, block_rows), :],
        sem,
    ).wait()


def kernel(embedding_table, label_indices):
    nc, d = embedding_table.shape
    n = int(label_indices.shape[0])

    idx = jnp.clip(label_indices.astype(jnp.int32), 0, nc - 1)

    block_rows = min(_BLOCK_ROWS, _round_up(max(n, 1), 8))
    n_pad = _round_up(max(n, 1), block_rows)
    if n_pad != n:
        idx = jnp.pad(idx, (0, n_pad - n))

    gather_fn = functools.partial(_gather_block_kernel, block_rows=block_rows)
    grid_spec = pltpu.PrefetchScalarGridSpec(
        num_scalar_prefetch=1,
        grid=(n_pad // block_rows,),
        in_specs=[pl.BlockSpec(memory_space=pl.ANY)],
        out_specs=pl.BlockSpec((block_rows, d), lambda i, idx_ref: (i, 0)),
        scratch_shapes=[pltpu.SemaphoreType.DMA],
    )
    out = pl.pallas_call(
        gather_fn,
        out_shape=jax.ShapeDtypeStruct((n_pad, d), embedding_table.dtype),
        grid_spec=grid_spec,
        compiler_params=pltpu.CompilerParams(
            dimension_semantics=("parallel",),
        ),
    )(idx, embedding_table)
    return out[:n]


# speedup vs baseline: 2.0848x; 1.2463x over previous
import functools

import jax
import jax.numpy as jnp
from jax.experimental import pallas as pl
from jax.experimental.pallas import tpu as pltpu

_BLOCK_ROWS = 256


def _round_up(x: int, m: int) -> int:
    return ((x + m - 1) // m) * m


def _gather_block_kernel(idx_ref, table_hbm, out_ref, sem, *, block_rows):
    base = pl.program_id(0) * block_rows
    for r in range(block_rows):
        row = idx_ref[base + r]
        pltpu.make_async_copy(
            table_hbm.at[pl.ds(row, 1), :],
            out_ref.at[pl.ds(r, 1), :],
            sem,
        ).start()
    pltpu.make_async_copy(
        table_hbm.at[pl.ds(0, block_rows), :],
        out_ref.at[pl.ds(0, block_rows), :],
        sem,
    ).wait()


def kernel(embedding_table, label_indices):
    nc, d = embedding_table.shape
    n = int(label_indices.shape[0])

    idx = jnp.clip(label_indices.astype(jnp.int32), 0, nc - 1)

    block_rows = min(_BLOCK_ROWS, _round_up(max(n, 1), 8))
    n_pad = _round_up(max(n, 1), block_rows)
    if n_pad != n:
        idx = jnp.pad(idx, (0, n_pad - n))

    gather_fn = functools.partial(_gather_block_kernel, block_rows=block_rows)
    grid_spec = pltpu.PrefetchScalarGridSpec(
        num_scalar_prefetch=1,
        grid=(n_pad // block_rows,),
        in_specs=[pl.BlockSpec(memory_space=pl.ANY)],
        out_specs=pl.BlockSpec((block_rows, d), lambda i, idx_ref: (i, 0)),
        scratch_shapes=[pltpu.SemaphoreType.DMA],
    )
    out = pl.pallas_call(
        gather_fn,
        out_shape=jax.ShapeDtypeStruct((n_pad, d), embedding_table.dtype),
        grid_spec=grid_spec,
        compiler_params=pltpu.CompilerParams(
            dimension_semantics=("parallel",),
        ),
    )(idx, embedding_table)
    return out[:n]


# speedup vs baseline: 2.4128x; 1.1573x over previous
import functools

import jax
import jax.numpy as jnp
from jax.experimental import pallas as pl
from jax.experimental.pallas import tpu as pltpu

_BLOCK_ROWS = 512


def _round_up(x: int, m: int) -> int:
    return ((x + m - 1) // m) * m


def _gather_block_kernel(idx_ref, table_hbm, out_ref, sem, *, block_rows):
    base = pl.program_id(0) * block_rows
    for r in range(block_rows):
        row = idx_ref[base + r]
        pltpu.make_async_copy(
            table_hbm.at[pl.ds(row, 1), :],
            out_ref.at[pl.ds(r, 1), :],
            sem,
        ).start()
    pltpu.make_async_copy(
        table_hbm.at[pl.ds(0, block_rows), :],
        out_ref.at[pl.ds(0, block_rows), :],
        sem,
    ).wait()


def kernel(embedding_table, label_indices):
    nc, d = embedding_table.shape
    n = int(label_indices.shape[0])

    idx = jnp.clip(label_indices.astype(jnp.int32), 0, nc - 1)

    block_rows = min(_BLOCK_ROWS, _round_up(max(n, 1), 8))
    n_pad = _round_up(max(n, 1), block_rows)
    if n_pad != n:
        idx = jnp.pad(idx, (0, n_pad - n))

    gather_fn = functools.partial(_gather_block_kernel, block_rows=block_rows)
    grid_spec = pltpu.PrefetchScalarGridSpec(
        num_scalar_prefetch=1,
        grid=(n_pad // block_rows,),
        in_specs=[pl.BlockSpec(memory_space=pl.ANY)],
        out_specs=pl.BlockSpec((block_rows, d), lambda i, idx_ref: (i, 0)),
        scratch_shapes=[pltpu.SemaphoreType.DMA],
    )
    out = pl.pallas_call(
        gather_fn,
        out_shape=jax.ShapeDtypeStruct((n_pad, d), embedding_table.dtype),
        grid_spec=grid_spec,
        compiler_params=pltpu.CompilerParams(
            dimension_semantics=("parallel",),
        ),
    )(idx, embedding_table)
    return out[:n]
